# baseline (speedup 1.0000x reference)
import functools

import jax
import jax.numpy as jnp
from jax import lax
from jax.experimental import pallas as pl
from jax.experimental.pallas import tpu as pltpu
from jax.experimental.pallas import tpu_sc as plsc

_E = 64
_TOPK = 8
_NEG = -1e30
_TM = 128
_NU = 192
_NUP = 256
_NCH = 128
_CL = 128


def _silu(x):
    return x * jax.nn.sigmoid(x)


def _gate_kernel(h_ref, gw_ref, idx_ref, w_ref):
    h = h_ref[...]
    gw = gw_ref[...]
    logits = lax.dot_general(
        h, gw, (((1,), (1,)), ((), ())), preferred_element_type=jnp.float32
    )
    iota_e = lax.broadcasted_iota(jnp.int32, logits.shape, 1)

    s = logits
    idx_cols = []
    val_cols = []
    for _ in range(_TOPK):
        m = jnp.max(s, axis=-1, keepdims=True)
        amax = jnp.min(jnp.where(s == m, iota_e, _E), axis=-1, keepdims=True)
        idx_cols.append(amax)
        val_cols.append(m)
        s = jnp.where(iota_e == amax, _NEG, s)
    lk = jnp.concatenate(val_cols, axis=1)
    ik = jnp.concatenate(idx_cols, axis=1)
    ex = jnp.exp(lk - lk[:, :1])
    w = ex / jnp.sum(ex, axis=-1, keepdims=True)
    idx_ref[...] = ik
    w_ref[...] = w


def _doti(a, b):
    return jnp.dot(a, b, precision=jax.lax.Precision.HIGHEST,
                   preferred_element_type=jnp.float32)


def _meta_kernel(ids_ref, dest_ref, g_ref, m_ref, rs_ref, re_ref):
    NCE = _NCH * _E
    ids2d = ids_ref[...]
    ids = jnp.broadcast_to(
        ids2d[:, :, None], (_CL, _NCH, _E)
    ).reshape(_CL, NCE)
    e_pat = lax.broadcasted_iota(jnp.int32, (_CL, NCE), 1) & (_E - 1)
    O = (ids == e_pat).astype(jnp.float32)

    ri = lax.broadcasted_iota(jnp.int32, (_CL, _CL), 0)
    ci = lax.broadcasted_iota(jnp.int32, (_CL, _CL), 1)
    Lstrict = (ri > ci).astype(jnp.float32)
    R = _doti(Lstrict, O)

    S = jnp.sum(O, axis=0, keepdims=True)
    P = S
    sh = _E
    for _ in range(7):
        shifted = jnp.concatenate(
            [jnp.zeros((1, sh), jnp.float32), P[:, : NCE - sh]], axis=1
        )
        P = P + shifted
        sh *= 2
    Bex = P - S
    tot = lax.slice(P, (0, NCE - _E), (1, NCE))

    ge = lax.broadcasted_iota(jnp.int32, (_E, _E), 0)
    gc = lax.broadcasted_iota(jnp.int32, (_E, _E), 1)
    Mstrict = (ge < gc).astype(jnp.float32)
    offs = _doti(tot, Mstrict)

    q_row = lax.broadcasted_iota(jnp.int32, (_E, NCE), 0)
    q_e = lax.broadcasted_iota(jnp.int32, (_E, NCE), 1) & (_E - 1)
    Q = (q_row == q_e).astype(jnp.float32)
    offs_t = _doti(offs, Q)

    F = R + Bex + offs_t
    pos_m = O * F
    z_m = lax.broadcasted_iota(jnp.int32, (NCE, _NCH), 0) >> 6
    z_c = lax.broadcasted_iota(jnp.int32, (NCE, _NCH), 1)
    Z = (z_m == z_c).astype(jnp.float32)
    pos_ic = _doti(pos_m, Z)
    dest_ref[...] = pos_ic.astype(jnp.int32)

    offs_i = offs.astype(jnp.int32)
    tot_i = tot.astype(jnp.int32)
    gend_i = offs_i + tot_i
    ft = offs_i >> 7
    lt = (gend_i + (_TM - 1)) >> 7
    touched = jnp.where(tot_i > 0, lt - ft, 0)
    cumx = _doti(touched.astype(jnp.float32), Mstrict).astype(jnp.int32)
    ci_incl = cumx + touched

    eye = (ge == gc).astype(jnp.float32)
    ci_b = jnp.broadcast_to(ci_incl.astype(jnp.float32), (_E, _E))
    ones_u = jnp.ones((_E, _NUP), jnp.float32)
    ci_cols = _doti(eye * ci_b, ones_u)
    u_b = lax.broadcasted_iota(jnp.int32, (_E, _NUP), 1)
    gsel = (ci_cols.astype(jnp.int32) <= u_b).astype(jnp.float32)
    g_of_u = jnp.sum(gsel, axis=0, keepdims=True).astype(jnp.int32)

    goh = (
        lax.broadcasted_iota(jnp.int32, (_E, _NUP), 0) == g_of_u
    ).astype(jnp.float32)

    def pick(v):
        return _doti(v.astype(jnp.float32), goh).astype(jnp.int32)

    ft_u = pick(ft)
    cumx_u = pick(cumx)
    gs_u = pick(offs_i)
    gend_u = pick(gend_i)
    u_iota = lax.broadcasted_iota(jnp.int32, (1, _NUP), 1)
    valid = g_of_u < _E
    unit_m = jnp.where(valid, ft_u + (u_iota - cumx_u), _NCH - 1)
    unit_g = jnp.minimum(g_of_u, _E - 1)
    rs = jnp.where(valid, jnp.maximum(gs_u, unit_m * _TM), 0)
    re = jnp.where(valid, jnp.minimum(gend_u, unit_m * _TM + _TM), 0)
    g_ref[...] = unit_g
    m_ref[...] = unit_m
    rs_ref[...] = rs
    re_ref[...] = re


def _make_dispatch(T, D, TD):
    mesh = plsc.VectorSubcoreMesh(core_axis_name="c", subcore_axis_name="s")

    @functools.partial(
        pl.kernel,
        mesh=mesh,
        out_type=(
            jax.ShapeDtypeStruct((TD, D), jnp.float32),
            jax.ShapeDtypeStruct((TD,), jnp.float32),
        ),
        scratch_types=[
            pltpu.VMEM((4, _CL), jnp.int32),
            pltpu.VMEM((4, _CL), jnp.int32),
            pltpu.VMEM((4, _CL), jnp.float32),
            pltpu.VMEM((_CL, D), jnp.float32),
            pltpu.SemaphoreType.DMA,
            pltpu.SemaphoreType.DMA,
        ],
    )
    def dispatch(h_hbm, dest_hbm, w_hbm, xs_hbm, ws_hbm,
                 dest_v, tok_v, w_v, xbuf, sem_e, sem_r):
        c = lax.axis_index("c")
        s = lax.axis_index("s")
        wid = s * 2 + c
        row0 = wid * 4
        pltpu.sync_copy(dest_hbm.at[pl.ds(row0, 4)], dest_v)
        pltpu.sync_copy(w_hbm.at[pl.ds(row0, 4)], w_v)
        base = wid * 512
        for j in range(4):
            for v in range(8):
                tok_v[j, pl.ds(v * 16, 16)] = (
                    base + j * _CL + v * 16 + lax.iota(jnp.int32, 16)
                ) >> 3
        pending = []
        for j in range(4):
            pending.append(
                pltpu.async_copy(w_v.at[j], ws_hbm.at[dest_v.at[j]], sem_e)
            )
        for j in range(4):
            pltpu.async_copy(h_hbm.at[tok_v.at[j]], xbuf, sem_r).wait()
            pltpu.async_copy(xbuf, xs_hbm.at[dest_v.at[j]], sem_r).wait()
        for p in pending:
            p.wait()

    return dispatch


def _ffn_kernel(g_sc, m_sc, rs_sc, re_sc,
                x_ref, w_ref, wg_ref, wu_ref, wd_ref, out_ref):
    u = pl.program_id(0)
    rs = rs_sc[u]
    re = re_sc[u]
    m = m_sc[u]
    row = m * _TM + lax.broadcasted_iota(jnp.int32, (_TM, 1), 0)
    valid = (row >= rs) & (row < re)
    x = x_ref[...]
    g = jnp.dot(x, wg_ref[0], preferred_element_type=jnp.float32)
    uu = jnp.dot(x, wu_ref[0], preferred_element_type=jnp.float32)
    y = jnp.dot(_silu(g) * uu, wd_ref[0], preferred_element_type=jnp.float32)
    ri = lax.broadcasted_iota(jnp.int32, (_TM, _TM), 0)
    ci = lax.broadcasted_iota(jnp.int32, (_TM, _TM), 1)
    wdiag = (ri == ci).astype(jnp.float32) * w_ref[0]
    wcol = _doti(wdiag, jnp.ones((_TM, 1), jnp.float32))
    w = jnp.where(valid, wcol, 0.0)
    yw = y * w
    prev_m = m_sc[jnp.maximum(u - 1, 0)]
    first = jnp.logical_or(u == 0, m != prev_m)

    @pl.when(first)
    def _():
        out_ref[...] = yw

    @pl.when(jnp.logical_not(first))
    def _():
        out_ref[...] += yw


def _make_combine(T, D, TD):
    mesh = plsc.VectorSubcoreMesh(core_axis_name="c", subcore_axis_name="s")

    @functools.partial(
        pl.kernel,
        mesh=mesh,
        out_type=jax.ShapeDtypeStruct((T, D), jnp.float32),
        scratch_types=[
            pltpu.VMEM((8, 64), jnp.int32),
            pltpu.VMEM((64, D), jnp.float32),
            pltpu.VMEM((64, D), jnp.float32),
            pltpu.SemaphoreType.DMA,
        ],
    )
    def combine(y_hbm, dest_hbm, shared_hbm, out_hbm, dest_v, ybuf, obuf,
                sem):
        c = lax.axis_index("c")
        s = lax.axis_index("s")
        wid = s * 2 + c
        row0 = wid * 8
        pltpu.sync_copy(dest_hbm.at[pl.ds(row0, 8)], dest_v)
        pltpu.sync_copy(shared_hbm.at[pl.ds(wid * 64, 64)], obuf)
        for j in range(8):
            pltpu.async_copy(y_hbm.at[dest_v.at[j]], ybuf, sem).wait()

            def body(v, _, j=j):
                col = v * 16
                for tl in range(8):
                    r = tl * 8
                    acc = ybuf[r, pl.ds(col, 16)]
                    for q in range(1, 8):
                        acc = acc + ybuf[r + q, pl.ds(col, 16)]
                    o = j * 8 + tl
                    obuf[o, pl.ds(col, 16)] = obuf[o, pl.ds(col, 16)] + acc
                return 0

            lax.fori_loop(0, D // 16, body, 0)
        pltpu.sync_copy(obuf, out_hbm.at[pl.ds(wid * 64, 64)])

    return combine


def _shared_kernel(h_ref, wsg_ref, wsu_ref, wsd_ref, out_ref):
    x = h_ref[...]
    g = jnp.dot(x, wsg_ref[...], preferred_element_type=jnp.float32)
    u = jnp.dot(x, wsu_ref[...], preferred_element_type=jnp.float32)
    y = jnp.dot(_silu(g) * u, wsd_ref[...], preferred_element_type=jnp.float32)
    out_ref[...] = y


def kernel(hidden_states, gate_weight, w_gate, w_up, w_down, ws_gate, ws_up, ws_down):
    orig_shape = hidden_states.shape
    D = orig_shape[-1]
    h = hidden_states.reshape(-1, D)
    T = h.shape[0]
    TD = T * _TOPK
    E, _, F = w_gate.shape

    idx, w = pl.pallas_call(
        _gate_kernel,
        out_shape=(
            jax.ShapeDtypeStruct((T, _TOPK), jnp.int32),
            jax.ShapeDtypeStruct((T, _TOPK), jnp.float32),
        ),
    )(h, gate_weight)

    ids2d = idx.reshape(_CL, _NCH)

    dest_ic, g_u, m_u, rs_u, re_u = pl.pallas_call(
        _meta_kernel,
        out_shape=(
            jax.ShapeDtypeStruct((_CL, _NCH), jnp.int32),
            jax.ShapeDtypeStruct((1, _NUP), jnp.int32),
            jax.ShapeDtypeStruct((1, _NUP), jnp.int32),
            jax.ShapeDtypeStruct((1, _NUP), jnp.int32),
            jax.ShapeDtypeStruct((1, _NUP), jnp.int32),
        ),
    )(ids2d)

    dest_ci = dest_ic
    w2d = w.reshape(_CL, _NCH)

    TT = 512
    shared = pl.pallas_call(
        _shared_kernel,
        grid=(T // TT,),
        in_specs=[
            pl.BlockSpec((TT, D), lambda t: (t, 0)),
            pl.BlockSpec(ws_gate.shape, lambda t: (0, 0)),
            pl.BlockSpec(ws_up.shape, lambda t: (0, 0)),
            pl.BlockSpec(ws_down.shape, lambda t: (0, 0)),
        ],
        out_specs=pl.BlockSpec((TT, D), lambda t: (t, 0)),
        out_shape=jax.ShapeDtypeStruct((T, D), jnp.float32),
    )(h, ws_gate, ws_up, ws_down)

    x_sorted, w_sorted = _make_dispatch(T, D, TD)(h, dest_ci, w2d)

    g_u = g_u.reshape(_NUP)[:_NU]
    m_u = m_u.reshape(_NUP)[:_NU]
    rs_u = rs_u.reshape(_NUP)[:_NU]
    re_u = re_u.reshape(_NUP)[:_NU]
    w_col = w_sorted.reshape(TD // _TM, 1, _TM)

    grid_spec = pltpu.PrefetchScalarGridSpec(
        num_scalar_prefetch=4,
        grid=(_NU,),
        in_specs=[
            pl.BlockSpec((_TM, D), lambda u, g, m, rs, re: (m[u], 0)),
            pl.BlockSpec((1, 1, _TM), lambda u, g, m, rs, re: (m[u], 0, 0)),
            pl.BlockSpec((1, D, F), lambda u, g, m, rs, re: (g[u], 0, 0)),
            pl.BlockSpec((1, D, F), lambda u, g, m, rs, re: (g[u], 0, 0)),
            pl.BlockSpec((1, F, D), lambda u, g, m, rs, re: (g[u], 0, 0)),
        ],
        out_specs=pl.BlockSpec((_TM, D), lambda u, g, m, rs, re: (m[u], 0)),
    )
    y_sorted = pl.pallas_call(
        _ffn_kernel,
        grid_spec=grid_spec,
        out_shape=jax.ShapeDtypeStruct((TD, D), jnp.float32),
    )(g_u, m_u, rs_u, re_u, x_sorted, w_col, w_gate, w_up, w_down)

    out = _make_combine(T, D, TD)(
        y_sorted, dest_ci.reshape(TD // 64, 64), shared
    )

    return out.reshape(orig_shape)


# speedup vs baseline: 1.0127x; 1.0127x over previous
import functools

import jax
import jax.numpy as jnp
from jax import lax
from jax.experimental import pallas as pl
from jax.experimental.pallas import tpu as pltpu
from jax.experimental.pallas import tpu_sc as plsc

_E = 64
_TOPK = 8
_NEG = -1e30
_TM = 128
_NU = 192
_NUP = 256
_NCH = 128
_CL = 128


def _silu(x):
    return x * jax.nn.sigmoid(x)


def _gate_kernel(h_ref, gw_ref, idx_ref, w_ref):
    h = h_ref[...]
    gw = gw_ref[...]
    logits = lax.dot_general(
        h, gw, (((1,), (1,)), ((), ())), preferred_element_type=jnp.float32
    )
    iota_e = lax.broadcasted_iota(jnp.int32, logits.shape, 1)

    s = logits
    idx_cols = []
    val_cols = []
    for _ in range(_TOPK):
        m = jnp.max(s, axis=-1, keepdims=True)
        amax = jnp.min(jnp.where(s == m, iota_e, _E), axis=-1, keepdims=True)
        idx_cols.append(amax)
        val_cols.append(m)
        s = jnp.where(iota_e == amax, _NEG, s)
    lk = jnp.concatenate(val_cols, axis=1)
    ik = jnp.concatenate(idx_cols, axis=1)
    ex = jnp.exp(lk - lk[:, :1])
    w = ex / jnp.sum(ex, axis=-1, keepdims=True)
    idx_ref[...] = ik
    w_ref[...] = w


def _doti(a, b):
    return jnp.dot(a, b, precision=jax.lax.Precision.HIGHEST,
                   preferred_element_type=jnp.float32)


def _meta_kernel(ids_ref, dest_ref, g_ref, m_ref, rs_ref, re_ref):
    NCE = _NCH * _E
    ids2d = ids_ref[...]
    ids = jnp.broadcast_to(
        ids2d[:, :, None], (_CL, _NCH, _E)
    ).reshape(_CL, NCE)
    e_pat = lax.broadcasted_iota(jnp.int32, (_CL, NCE), 1) & (_E - 1)
    O = (ids == e_pat).astype(jnp.float32)

    ri = lax.broadcasted_iota(jnp.int32, (_CL, _CL), 0)
    ci = lax.broadcasted_iota(jnp.int32, (_CL, _CL), 1)
    Lstrict = (ri > ci).astype(jnp.float32)
    R = _doti(Lstrict, O)

    S = jnp.sum(O, axis=0, keepdims=True)
    P = S
    sh = _E
    for _ in range(7):
        shifted = jnp.concatenate(
            [jnp.zeros((1, sh), jnp.float32), P[:, : NCE - sh]], axis=1
        )
        P = P + shifted
        sh *= 2
    Bex = P - S
    tot = lax.slice(P, (0, NCE - _E), (1, NCE))

    ge = lax.broadcasted_iota(jnp.int32, (_E, _E), 0)
    gc = lax.broadcasted_iota(jnp.int32, (_E, _E), 1)
    Mstrict = (ge < gc).astype(jnp.float32)
    offs = _doti(tot, Mstrict)

    q_row = lax.broadcasted_iota(jnp.int32, (_E, NCE), 0)
    q_e = lax.broadcasted_iota(jnp.int32, (_E, NCE), 1) & (_E - 1)
    Q = (q_row == q_e).astype(jnp.float32)
    offs_t = _doti(offs, Q)

    F = R + Bex + offs_t
    pos_m = O * F
    z_m = lax.broadcasted_iota(jnp.int32, (NCE, _NCH), 0) >> 6
    z_c = lax.broadcasted_iota(jnp.int32, (NCE, _NCH), 1)
    Z = (z_m == z_c).astype(jnp.float32)
    pos_ic = _doti(pos_m, Z)
    dest_ref[...] = pos_ic.astype(jnp.int32)

    offs_i = offs.astype(jnp.int32)
    tot_i = tot.astype(jnp.int32)
    gend_i = offs_i + tot_i
    ft = offs_i >> 7
    lt = (gend_i + (_TM - 1)) >> 7
    touched = jnp.where(tot_i > 0, lt - ft, 0)
    cumx = _doti(touched.astype(jnp.float32), Mstrict).astype(jnp.int32)
    ci_incl = cumx + touched

    eye = (ge == gc).astype(jnp.float32)
    ci_b = jnp.broadcast_to(ci_incl.astype(jnp.float32), (_E, _E))
    ones_u = jnp.ones((_E, _NUP), jnp.float32)
    ci_cols = _doti(eye * ci_b, ones_u)
    u_b = lax.broadcasted_iota(jnp.int32, (_E, _NUP), 1)
    gsel = (ci_cols.astype(jnp.int32) <= u_b).astype(jnp.float32)
    g_of_u = jnp.sum(gsel, axis=0, keepdims=True).astype(jnp.int32)

    goh = (
        lax.broadcasted_iota(jnp.int32, (_E, _NUP), 0) == g_of_u
    ).astype(jnp.float32)

    def pick(v):
        return _doti(v.astype(jnp.float32), goh).astype(jnp.int32)

    ft_u = pick(ft)
    cumx_u = pick(cumx)
    gs_u = pick(offs_i)
    gend_u = pick(gend_i)
    u_iota = lax.broadcasted_iota(jnp.int32, (1, _NUP), 1)
    valid = g_of_u < _E
    unit_m = jnp.where(valid, ft_u + (u_iota - cumx_u), _NCH - 1)
    unit_g = jnp.minimum(g_of_u, _E - 1)
    rs = jnp.where(valid, jnp.maximum(gs_u, unit_m * _TM), 0)
    re = jnp.where(valid, jnp.minimum(gend_u, unit_m * _TM + _TM), 0)
    g_ref[...] = unit_g
    m_ref[...] = unit_m
    rs_ref[...] = rs
    re_ref[...] = re


def _make_dispatch(T, D, TD):
    mesh = plsc.VectorSubcoreMesh(core_axis_name="c", subcore_axis_name="s")

    @functools.partial(
        pl.kernel,
        mesh=mesh,
        out_type=(
            jax.ShapeDtypeStruct((TD, D), jnp.float32),
            jax.ShapeDtypeStruct((TD,), jnp.float32),
        ),
        scratch_types=[
            pltpu.VMEM((8, 64), jnp.int32),
            pltpu.VMEM((8, 64), jnp.int32),
            pltpu.VMEM((8, 64), jnp.float32),
            pltpu.VMEM((64, D), jnp.float32),
            pltpu.VMEM((64, D), jnp.float32),
            pltpu.SemaphoreType.DMA,
            pltpu.SemaphoreType.DMA,
            pltpu.SemaphoreType.DMA,
        ],
    )
    def dispatch(h_hbm, dest_hbm, w_hbm, xs_hbm, ws_hbm,
                 dest_v, tok_v, w_v, xbufa, xbufb, sem_e, sem_g, sem_s):
        c = lax.axis_index("c")
        s = lax.axis_index("s")
        wid = s * 2 + c
        row0 = wid * 8
        pltpu.sync_copy(dest_hbm.at[pl.ds(row0, 8)], dest_v)
        pltpu.sync_copy(w_hbm.at[pl.ds(row0, 8)], w_v)
        base = wid * 512
        for j in range(8):
            for v in range(4):
                tok_v[j, pl.ds(v * 16, 16)] = (
                    base + j * 64 + v * 16 + lax.iota(jnp.int32, 16)
                ) >> 3
        pending = []
        for j in range(8):
            pending.append(
                pltpu.async_copy(w_v.at[j], ws_hbm.at[dest_v.at[j]], sem_e)
            )
        bufs = [xbufa, xbufb]
        g = pltpu.async_copy(h_hbm.at[tok_v.at[0]], bufs[0], sem_g)
        sc_prev = None
        for j in range(8):
            g.wait()
            if sc_prev is not None:
                sc_prev.wait()
            if j + 1 < 8:
                g = pltpu.async_copy(
                    h_hbm.at[tok_v.at[j + 1]], bufs[(j + 1) % 2], sem_g
                )
            sc_prev = pltpu.async_copy(
                bufs[j % 2], xs_hbm.at[dest_v.at[j]], sem_s
            )
        sc_prev.wait()
        for p in pending:
            p.wait()

    return dispatch


def _ffn_kernel(g_sc, m_sc, rs_sc, re_sc,
                x_ref, w_ref, wg_ref, wu_ref, wd_ref, out_ref):
    u = pl.program_id(0)
    rs = rs_sc[u]
    re = re_sc[u]
    m = m_sc[u]
    row = m * _TM + lax.broadcasted_iota(jnp.int32, (_TM, 1), 0)
    valid = (row >= rs) & (row < re)
    x = x_ref[...]
    g = jnp.dot(x, wg_ref[0], preferred_element_type=jnp.float32)
    uu = jnp.dot(x, wu_ref[0], preferred_element_type=jnp.float32)
    y = jnp.dot(_silu(g) * uu, wd_ref[0], preferred_element_type=jnp.float32)
    ri = lax.broadcasted_iota(jnp.int32, (_TM, _TM), 0)
    ci = lax.broadcasted_iota(jnp.int32, (_TM, _TM), 1)
    wdiag = (ri == ci).astype(jnp.float32) * w_ref[0]
    wcol = _doti(wdiag, jnp.ones((_TM, 1), jnp.float32))
    w = jnp.where(valid, wcol, 0.0)
    yw = y * w
    prev_m = m_sc[jnp.maximum(u - 1, 0)]
    first = jnp.logical_or(u == 0, m != prev_m)

    @pl.when(first)
    def _():
        out_ref[...] = yw

    @pl.when(jnp.logical_not(first))
    def _():
        out_ref[...] += yw


def _make_combine(T, D, TD):
    mesh = plsc.VectorSubcoreMesh(core_axis_name="c", subcore_axis_name="s")

    @functools.partial(
        pl.kernel,
        mesh=mesh,
        out_type=jax.ShapeDtypeStruct((T, D), jnp.float32),
        scratch_types=[
            pltpu.VMEM((8, 64), jnp.int32),
            pltpu.VMEM((64, D), jnp.float32),
            pltpu.VMEM((64, D), jnp.float32),
            pltpu.SemaphoreType.DMA,
        ],
    )
    def combine(y_hbm, dest_hbm, shared_hbm, out_hbm, dest_v, ybuf, obuf,
                sem):
        c = lax.axis_index("c")
        s = lax.axis_index("s")
        wid = s * 2 + c
        row0 = wid * 8
        pltpu.sync_copy(dest_hbm.at[pl.ds(row0, 8)], dest_v)
        pltpu.sync_copy(shared_hbm.at[pl.ds(wid * 64, 64)], obuf)
        for j in range(8):
            pltpu.async_copy(y_hbm.at[dest_v.at[j]], ybuf, sem).wait()

            def body(v, _, j=j):
                col = v * 16
                for tl in range(8):
                    r = tl * 8
                    acc = ybuf[r, pl.ds(col, 16)]
                    for q in range(1, 8):
                        acc = acc + ybuf[r + q, pl.ds(col, 16)]
                    o = j * 8 + tl
                    obuf[o, pl.ds(col, 16)] = obuf[o, pl.ds(col, 16)] + acc
                return 0

            lax.fori_loop(0, D // 16, body, 0)
        pltpu.sync_copy(obuf, out_hbm.at[pl.ds(wid * 64, 64)])

    return combine


def _shared_kernel(h_ref, wsg_ref, wsu_ref, wsd_ref, out_ref):
    x = h_ref[...]
    g = jnp.dot(x, wsg_ref[...], preferred_element_type=jnp.float32)
    u = jnp.dot(x, wsu_ref[...], preferred_element_type=jnp.float32)
    y = jnp.dot(_silu(g) * u, wsd_ref[...], preferred_element_type=jnp.float32)
    out_ref[...] = y


def kernel(hidden_states, gate_weight, w_gate, w_up, w_down, ws_gate, ws_up, ws_down):
    orig_shape = hidden_states.shape
    D = orig_shape[-1]
    h = hidden_states.reshape(-1, D)
    T = h.shape[0]
    TD = T * _TOPK
    E, _, F = w_gate.shape

    idx, w = pl.pallas_call(
        _gate_kernel,
        out_shape=(
            jax.ShapeDtypeStruct((T, _TOPK), jnp.int32),
            jax.ShapeDtypeStruct((T, _TOPK), jnp.float32),
        ),
    )(h, gate_weight)

    ids2d = idx.reshape(_CL, _NCH)

    dest_ic, g_u, m_u, rs_u, re_u = pl.pallas_call(
        _meta_kernel,
        out_shape=(
            jax.ShapeDtypeStruct((_CL, _NCH), jnp.int32),
            jax.ShapeDtypeStruct((1, _NUP), jnp.int32),
            jax.ShapeDtypeStruct((1, _NUP), jnp.int32),
            jax.ShapeDtypeStruct((1, _NUP), jnp.int32),
            jax.ShapeDtypeStruct((1, _NUP), jnp.int32),
        ),
    )(ids2d)

    dest_ci = dest_ic
    w2d = w.reshape(_CL, _NCH)

    TT = 512
    shared = pl.pallas_call(
        _shared_kernel,
        grid=(T // TT,),
        in_specs=[
            pl.BlockSpec((TT, D), lambda t: (t, 0)),
            pl.BlockSpec(ws_gate.shape, lambda t: (0, 0)),
            pl.BlockSpec(ws_up.shape, lambda t: (0, 0)),
            pl.BlockSpec(ws_down.shape, lambda t: (0, 0)),
        ],
        out_specs=pl.BlockSpec((TT, D), lambda t: (t, 0)),
        out_shape=jax.ShapeDtypeStruct((T, D), jnp.float32),
    )(h, ws_gate, ws_up, ws_down)

    x_sorted, w_sorted = _make_dispatch(T, D, TD)(
        h, dest_ci.reshape(TD // 64, 64), w2d.reshape(TD // 64, 64)
    )

    g_u = g_u.reshape(_NUP)[:_NU]
    m_u = m_u.reshape(_NUP)[:_NU]
    rs_u = rs_u.reshape(_NUP)[:_NU]
    re_u = re_u.reshape(_NUP)[:_NU]
    w_col = w_sorted.reshape(TD // _TM, 1, _TM)

    grid_spec = pltpu.PrefetchScalarGridSpec(
        num_scalar_prefetch=4,
        grid=(_NU,),
        in_specs=[
            pl.BlockSpec((_TM, D), lambda u, g, m, rs, re: (m[u], 0)),
            pl.BlockSpec((1, 1, _TM), lambda u, g, m, rs, re: (m[u], 0, 0)),
            pl.BlockSpec((1, D, F), lambda u, g, m, rs, re: (g[u], 0, 0)),
            pl.BlockSpec((1, D, F), lambda u, g, m, rs, re: (g[u], 0, 0)),
            pl.BlockSpec((1, F, D), lambda u, g, m, rs, re: (g[u], 0, 0)),
        ],
        out_specs=pl.BlockSpec((_TM, D), lambda u, g, m, rs, re: (m[u], 0)),
    )
    y_sorted = pl.pallas_call(
        _ffn_kernel,
        grid_spec=grid_spec,
        out_shape=jax.ShapeDtypeStruct((TD, D), jnp.float32),
    )(g_u, m_u, rs_u, re_u, x_sorted, w_col, w_gate, w_up, w_down)

    out = _make_combine(T, D, TD)(
        y_sorted, dest_ci.reshape(TD // 64, 64), shared
    )

    return out.reshape(orig_shape)


# speedup vs baseline: 1.0451x; 1.0320x over previous
import functools

import jax
import jax.numpy as jnp
from jax import lax
from jax.experimental import pallas as pl
from jax.experimental.pallas import tpu as pltpu
from jax.experimental.pallas import tpu_sc as plsc

_E = 64
_TOPK = 8
_NEG = -1e30
_TM = 256
_NT = 64
_NU = 128
_NUP = 256
_NCH = 128
_CL = 128


def _silu(x):
    return x * jax.nn.sigmoid(x)


def _gate_kernel(h_ref, gw_ref, idx_ref, w_ref):
    h = h_ref[...]
    gw = gw_ref[...]
    logits = lax.dot_general(
        h, gw, (((1,), (1,)), ((), ())), preferred_element_type=jnp.float32
    )
    iota_e = lax.broadcasted_iota(jnp.int32, logits.shape, 1)

    s = logits
    idx_cols = []
    val_cols = []
    for _ in range(_TOPK):
        m = jnp.max(s, axis=-1, keepdims=True)
        amax = jnp.min(jnp.where(s == m, iota_e, _E), axis=-1, keepdims=True)
        idx_cols.append(amax)
        val_cols.append(m)
        s = jnp.where(iota_e == amax, _NEG, s)
    lk = jnp.concatenate(val_cols, axis=1)
    ik = jnp.concatenate(idx_cols, axis=1)
    ex = jnp.exp(lk - lk[:, :1])
    w = ex / jnp.sum(ex, axis=-1, keepdims=True)
    idx_ref[...] = ik
    w_ref[...] = w


def _doti(a, b):
    return jnp.dot(a, b, precision=jax.lax.Precision.HIGHEST,
                   preferred_element_type=jnp.float32)


def _meta_kernel(ids_ref, dest_ref, g_ref, m_ref, rs_ref, re_ref):
    NCE = _NCH * _E
    ids2d = ids_ref[...]
    ids = jnp.broadcast_to(
        ids2d[:, :, None], (_CL, _NCH, _E)
    ).reshape(_CL, NCE)
    e_pat = lax.broadcasted_iota(jnp.int32, (_CL, NCE), 1) & (_E - 1)
    O = (ids == e_pat).astype(jnp.float32)

    ri = lax.broadcasted_iota(jnp.int32, (_CL, _CL), 0)
    ci = lax.broadcasted_iota(jnp.int32, (_CL, _CL), 1)
    Lstrict = (ri > ci).astype(jnp.float32)
    R = _doti(Lstrict, O)

    S = jnp.sum(O, axis=0, keepdims=True)
    P = S
    sh = _E
    for _ in range(7):
        shifted = jnp.concatenate(
            [jnp.zeros((1, sh), jnp.float32), P[:, : NCE - sh]], axis=1
        )
        P = P + shifted
        sh *= 2
    Bex = P - S
    tot = lax.slice(P, (0, NCE - _E), (1, NCE))

    ge = lax.broadcasted_iota(jnp.int32, (_E, _E), 0)
    gc = lax.broadcasted_iota(jnp.int32, (_E, _E), 1)
    Mstrict = (ge < gc).astype(jnp.float32)
    offs = _doti(tot, Mstrict)

    q_row = lax.broadcasted_iota(jnp.int32, (_E, NCE), 0)
    q_e = lax.broadcasted_iota(jnp.int32, (_E, NCE), 1) & (_E - 1)
    Q = (q_row == q_e).astype(jnp.float32)
    offs_t = _doti(offs, Q)

    F = R + Bex + offs_t
    pos_m = O * F
    z_m = lax.broadcasted_iota(jnp.int32, (NCE, _NCH), 0) >> 6
    z_c = lax.broadcasted_iota(jnp.int32, (NCE, _NCH), 1)
    Z = (z_m == z_c).astype(jnp.float32)
    pos_ic = _doti(pos_m, Z)
    dest_ref[...] = pos_ic.astype(jnp.int32)

    offs_i = offs.astype(jnp.int32)
    tot_i = tot.astype(jnp.int32)
    gend_i = offs_i + tot_i
    ft = offs_i >> 8
    lt = (gend_i + (_TM - 1)) >> 8
    touched = jnp.where(tot_i > 0, lt - ft, 0)
    cumx = _doti(touched.astype(jnp.float32), Mstrict).astype(jnp.int32)
    ci_incl = cumx + touched

    eye = (ge == gc).astype(jnp.float32)
    ci_b = jnp.broadcast_to(ci_incl.astype(jnp.float32), (_E, _E))
    ones_u = jnp.ones((_E, _NUP), jnp.float32)
    ci_cols = _doti(eye * ci_b, ones_u)
    u_b = lax.broadcasted_iota(jnp.int32, (_E, _NUP), 1)
    gsel = (ci_cols.astype(jnp.int32) <= u_b).astype(jnp.float32)
    g_of_u = jnp.sum(gsel, axis=0, keepdims=True).astype(jnp.int32)

    goh = (
        lax.broadcasted_iota(jnp.int32, (_E, _NUP), 0) == g_of_u
    ).astype(jnp.float32)

    def pick(v):
        return _doti(v.astype(jnp.float32), goh).astype(jnp.int32)

    ft_u = pick(ft)
    cumx_u = pick(cumx)
    gs_u = pick(offs_i)
    gend_u = pick(gend_i)
    u_iota = lax.broadcasted_iota(jnp.int32, (1, _NUP), 1)
    valid = g_of_u < _E
    unit_m = jnp.where(valid, ft_u + (u_iota - cumx_u), _NT - 1)
    unit_g = jnp.minimum(g_of_u, _E - 1)
    rs = jnp.where(valid, jnp.maximum(gs_u, unit_m * _TM), 0)
    re = jnp.where(valid, jnp.minimum(gend_u, unit_m * _TM + _TM), 0)
    g_ref[...] = unit_g
    m_ref[...] = unit_m
    rs_ref[...] = rs
    re_ref[...] = re


def _make_dispatch(T, D, TD):
    mesh = plsc.VectorSubcoreMesh(core_axis_name="c", subcore_axis_name="s")

    @functools.partial(
        pl.kernel,
        mesh=mesh,
        out_type=(
            jax.ShapeDtypeStruct((TD, D), jnp.float32),
            jax.ShapeDtypeStruct((TD,), jnp.float32),
        ),
        scratch_types=[
            pltpu.VMEM((8, 64), jnp.int32),
            pltpu.VMEM((8, 64), jnp.int32),
            pltpu.VMEM((8, 64), jnp.float32),
            pltpu.VMEM((64, D), jnp.float32),
            pltpu.VMEM((64, D), jnp.float32),
            pltpu.SemaphoreType.DMA,
            pltpu.SemaphoreType.DMA,
            pltpu.SemaphoreType.DMA,
        ],
    )
    def dispatch(h_hbm, dest_hbm, w_hbm, xs_hbm, ws_hbm,
                 dest_v, tok_v, w_v, xbufa, xbufb, sem_e, sem_g, sem_s):
        c = lax.axis_index("c")
        s = lax.axis_index("s")
        wid = s * 2 + c
        row0 = wid * 8
        pltpu.sync_copy(dest_hbm.at[pl.ds(row0, 8)], dest_v)
        pltpu.sync_copy(w_hbm.at[pl.ds(row0, 8)], w_v)
        base = wid * 512
        for j in range(8):
            for v in range(4):
                tok_v[j, pl.ds(v * 16, 16)] = (
                    base + j * 64 + v * 16 + lax.iota(jnp.int32, 16)
                ) >> 3
        pending = []
        for j in range(8):
            pending.append(
                pltpu.async_copy(w_v.at[j], ws_hbm.at[dest_v.at[j]], sem_e)
            )
        bufs = [xbufa, xbufb]
        g = pltpu.async_copy(h_hbm.at[tok_v.at[0]], bufs[0], sem_g)
        sc_prev = None
        for j in range(8):
            g.wait()
            if sc_prev is not None:
                sc_prev.wait()
            if j + 1 < 8:
                g = pltpu.async_copy(
                    h_hbm.at[tok_v.at[j + 1]], bufs[(j + 1) % 2], sem_g
                )
            sc_prev = pltpu.async_copy(
                bufs[j % 2], xs_hbm.at[dest_v.at[j]], sem_s
            )
        sc_prev.wait()
        for p in pending:
            p.wait()

    return dispatch


def _ffn_kernel(g_sc, m_sc, rs_sc, re_sc,
                x_ref, w_ref, wg_ref, wu_ref, wd_ref, out_ref):
    u = pl.program_id(0)
    rs = rs_sc[u]
    re = re_sc[u]
    m = m_sc[u]
    row = m * _TM + lax.broadcasted_iota(jnp.int32, (_TM, 1), 0)
    valid = (row >= rs) & (row < re)
    x = x_ref[...]
    g = jnp.dot(x, wg_ref[0], preferred_element_type=jnp.float32)
    uu = jnp.dot(x, wu_ref[0], preferred_element_type=jnp.float32)
    y = jnp.dot(_silu(g) * uu, wd_ref[0], preferred_element_type=jnp.float32)
    ri = lax.broadcasted_iota(jnp.int32, (_TM, _TM), 0)
    ci = lax.broadcasted_iota(jnp.int32, (_TM, _TM), 1)
    wdiag = (ri == ci).astype(jnp.float32) * w_ref[0]
    wcol = _doti(wdiag, jnp.ones((_TM, 1), jnp.float32))
    w = jnp.where(valid, wcol, 0.0)
    yw = y * w
    prev_m = m_sc[jnp.maximum(u - 1, 0)]
    first = jnp.logical_or(u == 0, m != prev_m)

    @pl.when(first)
    def _():
        out_ref[...] = yw

    @pl.when(jnp.logical_not(first))
    def _():
        out_ref[...] += yw


def _make_combine(T, D, TD):
    mesh = plsc.VectorSubcoreMesh(core_axis_name="c", subcore_axis_name="s")

    @functools.partial(
        pl.kernel,
        mesh=mesh,
        out_type=jax.ShapeDtypeStruct((T, D), jnp.float32),
        scratch_types=[
            pltpu.VMEM((8, 64), jnp.int32),
            pltpu.VMEM((64, D), jnp.float32),
            pltpu.VMEM((64, D), jnp.float32),
            pltpu.SemaphoreType.DMA,
        ],
    )
    def combine(y_hbm, dest_hbm, shared_hbm, out_hbm, dest_v, ybuf, obuf,
                sem):
        c = lax.axis_index("c")
        s = lax.axis_index("s")
        wid = s * 2 + c
        row0 = wid * 8
        pltpu.sync_copy(dest_hbm.at[pl.ds(row0, 8)], dest_v)
        pltpu.sync_copy(shared_hbm.at[pl.ds(wid * 64, 64)], obuf)
        for j in range(8):
            pltpu.async_copy(y_hbm.at[dest_v.at[j]], ybuf, sem).wait()

            def body(v, _, j=j):
                col = v * 16
                for tl in range(8):
                    r = tl * 8
                    acc = ybuf[r, pl.ds(col, 16)]
                    for q in range(1, 8):
                        acc = acc + ybuf[r + q, pl.ds(col, 16)]
                    o = j * 8 + tl
                    obuf[o, pl.ds(col, 16)] = obuf[o, pl.ds(col, 16)] + acc
                return 0

            lax.fori_loop(0, D // 16, body, 0)
        pltpu.sync_copy(obuf, out_hbm.at[pl.ds(wid * 64, 64)])

    return combine


def _shared_kernel(h_ref, wsg_ref, wsu_ref, wsd_ref, out_ref):
    x = h_ref[...]
    g = jnp.dot(x, wsg_ref[...], preferred_element_type=jnp.float32)
    u = jnp.dot(x, wsu_ref[...], preferred_element_type=jnp.float32)
    y = jnp.dot(_silu(g) * u, wsd_ref[...], preferred_element_type=jnp.float32)
    out_ref[...] = y


def kernel(hidden_states, gate_weight, w_gate, w_up, w_down, ws_gate, ws_up, ws_down):
    orig_shape = hidden_states.shape
    D = orig_shape[-1]
    h = hidden_states.reshape(-1, D)
    T = h.shape[0]
    TD = T * _TOPK
    E, _, F = w_gate.shape

    idx, w = pl.pallas_call(
        _gate_kernel,
        out_shape=(
            jax.ShapeDtypeStruct((T, _TOPK), jnp.int32),
            jax.ShapeDtypeStruct((T, _TOPK), jnp.float32),
        ),
    )(h, gate_weight)

    ids2d = idx.reshape(_CL, _NCH)

    dest_ic, g_u, m_u, rs_u, re_u = pl.pallas_call(
        _meta_kernel,
        out_shape=(
            jax.ShapeDtypeStruct((_CL, _NCH), jnp.int32),
            jax.ShapeDtypeStruct((1, _NUP), jnp.int32),
            jax.ShapeDtypeStruct((1, _NUP), jnp.int32),
            jax.ShapeDtypeStruct((1, _NUP), jnp.int32),
            jax.ShapeDtypeStruct((1, _NUP), jnp.int32),
        ),
    )(ids2d)

    dest_ci = dest_ic
    w2d = w.reshape(_CL, _NCH)

    TT = 512
    shared = pl.pallas_call(
        _shared_kernel,
        grid=(T // TT,),
        in_specs=[
            pl.BlockSpec((TT, D), lambda t: (t, 0)),
            pl.BlockSpec(ws_gate.shape, lambda t: (0, 0)),
            pl.BlockSpec(ws_up.shape, lambda t: (0, 0)),
            pl.BlockSpec(ws_down.shape, lambda t: (0, 0)),
        ],
        out_specs=pl.BlockSpec((TT, D), lambda t: (t, 0)),
        out_shape=jax.ShapeDtypeStruct((T, D), jnp.float32),
    )(h, ws_gate, ws_up, ws_down)

    x_sorted, w_sorted = _make_dispatch(T, D, TD)(
        h, dest_ci.reshape(TD // 64, 64), w2d.reshape(TD // 64, 64)
    )

    g_u = g_u.reshape(_NUP)[:_NU]
    m_u = m_u.reshape(_NUP)[:_NU]
    rs_u = rs_u.reshape(_NUP)[:_NU]
    re_u = re_u.reshape(_NUP)[:_NU]
    w_col = w_sorted.reshape(TD // _TM, 1, _TM)

    grid_spec = pltpu.PrefetchScalarGridSpec(
        num_scalar_prefetch=4,
        grid=(_NU,),
        in_specs=[
            pl.BlockSpec((_TM, D), lambda u, g, m, rs, re: (m[u], 0)),
            pl.BlockSpec((1, 1, _TM), lambda u, g, m, rs, re: (m[u], 0, 0)),
            pl.BlockSpec((1, D, F), lambda u, g, m, rs, re: (g[u], 0, 0)),
            pl.BlockSpec((1, D, F), lambda u, g, m, rs, re: (g[u], 0, 0)),
            pl.BlockSpec((1, F, D), lambda u, g, m, rs, re: (g[u], 0, 0)),
        ],
        out_specs=pl.BlockSpec((_TM, D), lambda u, g, m, rs, re: (m[u], 0)),
    )
    y_sorted = pl.pallas_call(
        _ffn_kernel,
        grid_spec=grid_spec,
        out_shape=jax.ShapeDtypeStruct((TD, D), jnp.float32),
    )(g_u, m_u, rs_u, re_u, x_sorted, w_col, w_gate, w_up, w_down)

    out = _make_combine(T, D, TD)(
        y_sorted, dest_ci.reshape(TD // 64, 64), shared
    )

    return out.reshape(orig_shape)
